# baseline (speedup 1.0000x reference)
import functools

import jax
import jax.numpy as jnp
from jax import lax
from jax.experimental import pallas as pl
from jax.experimental.pallas import tpu as pltpu
from jax.experimental.pallas import tpu_sc as plsc

VOCAB = 100000
DIM = 128
BATCH = 16384
LMBDA = 0.01

NC, NS = 2, 16
NW = NC * NS
BPW = BATCH // NW
NCHUNK = 32
CHUNK = BPW // NCHUNK


_mesh = plsc.VectorSubcoreMesh(core_axis_name="c", subcore_axis_name="s")


@functools.partial(
    pl.kernel,
    out_type=(
        jax.ShapeDtypeStruct((BATCH,), jnp.float32),
        jax.ShapeDtypeStruct((NW, DIM), jnp.float32),
    ),
    mesh=_mesh,
    scratch_types=[
        pltpu.VMEM((NCHUNK, CHUNK), jnp.int32),
        pltpu.VMEM((NCHUNK, CHUNK), jnp.int32),
        pltpu.VMEM((NCHUNK, CHUNK), jnp.int32),
        pltpu.VMEM((2 * CHUNK, DIM), jnp.float32),
        pltpu.VMEM((2 * CHUNK, DIM), jnp.float32),
        pltpu.VMEM((2 * CHUNK, DIM), jnp.float32),
        pltpu.VMEM((BPW,), jnp.float32),
        pltpu.VMEM((DIM,), jnp.float32),
        pltpu.SemaphoreType.DMA,
        pltpu.SemaphoreType.DMA,
    ],
)
def _sc_gather_score(x0, x1, x2, w_obj, w_task, w_grasp,
                     score_out, sq_out,
                     i0, i1, i2, hb, rb, tb, sv, sqv, sem0, sem1):
    c = lax.axis_index("c")
    s = lax.axis_index("s")
    wid = s * NC + c

    pltpu.sync_copy(x0.at[wid], i0)
    pltpu.sync_copy(x1.at[wid], i1)
    pltpu.sync_copy(x2.at[wid], i2)

    lanes = lax.iota(jnp.int32, 16)

    _gdn = lax.GatherDimensionNumbers(
        offset_dims=(), collapsed_slice_dims=(0,), start_index_map=(0,))

    def shuffle(v, idx):
        return lax.gather(v, idx[:, None], dimension_numbers=_gdn,
                          slice_sizes=(1,),
                          mode=lax.GatherScatterMode.PROMISE_IN_BOUNDS)

    def hsum(v):
        for shift in (1, 2, 4, 8):
            v = v + shuffle(v, lanes ^ shift)
        return v

    def issue(j, base, sem):
        pltpu.async_copy(w_obj.at[i0.at[j]], hb.at[pl.ds(base, CHUNK)], sem)
        pltpu.async_copy(w_task.at[i1.at[j]], rb.at[pl.ds(base, CHUNK)], sem)
        pltpu.async_copy(w_grasp.at[i2.at[j]], tb.at[pl.ds(base, CHUNK)], sem)

    def drain(base, sem):
        pltpu.make_async_copy(
            w_obj.at[i0.at[0]], hb.at[pl.ds(base, CHUNK)], sem).wait()
        pltpu.make_async_copy(
            w_task.at[i1.at[0]], rb.at[pl.ds(base, CHUNK)], sem).wait()
        pltpu.make_async_copy(
            w_grasp.at[i2.at[0]], tb.at[pl.ds(base, CHUNK)], sem).wait()

    issue(0, 0, sem0)

    def chunk_fn(j, sq8_in):
        slot = lax.rem(j, 2)
        base = slot * CHUNK

        @pl.when(j + 1 < NCHUNK)
        def _():
            @pl.when(slot == 0)
            def _():
                issue(j + 1, CHUNK, sem1)

            @pl.when(slot == 1)
            def _():
                issue(j + 1, 0, sem0)

        @pl.when(slot == 0)
        def _():
            drain(0, sem0)

        @pl.when(slot == 1)
        def _():
            drain(CHUNK, sem1)

        def blk_fn(b, sq8_):
            sq8_ = list(sq8_)
            scorevec = jnp.zeros((16,), jnp.float32)
            for l in range(16):
                r = base + b * 16 + l
                acc = jnp.zeros((16,), jnp.float32)
                for g in range(8):
                    hv = hb[r, pl.ds(g * 16, 16)]
                    rv = rb[r, pl.ds(g * 16, 16)]
                    tv = tb[r, pl.ds(g * 16, 16)]
                    acc = acc + hv * tv * rv
                    sq8_[g] = sq8_[g] + hv * hv + tv * tv + rv * rv
                scorevec = jnp.where(lanes == l, -hsum(acc), scorevec)
            sv[pl.ds(j * CHUNK + b * 16, 16)] = scorevec
            return tuple(sq8_)

        return lax.fori_loop(0, CHUNK // 16, blk_fn, sq8_in)

    sq8 = tuple(jnp.zeros((16,), jnp.float32) for _ in range(8))
    sq8 = lax.fori_loop(0, NCHUNK, chunk_fn, sq8)

    for g in range(8):
        sqv[pl.ds(g * 16, 16)] = sq8[g]
    pltpu.sync_copy(sqv, sq_out.at[wid])
    pltpu.sync_copy(sv, score_out.at[pl.ds(wid * BPW, BPW)])


def _tc_loss_body(score_ref, y_ref, sq_ref, out_ref):
    z = score_ref[...] * y_ref[...]
    sp = jnp.maximum(z, 0.0) + jnp.log1p(jnp.exp(-jnp.abs(z)))
    regul = jnp.sum(sq_ref[...]) * (1.0 / (BATCH * DIM))
    out_ref[0, 0] = jnp.sum(sp) * (1.0 / BATCH) + LMBDA * regul


def kernel(x, y, W_obj, W_task, W_grasp):
    xi = x.astype(jnp.int32)
    x0 = xi[:, 0].reshape(NW, NCHUNK, CHUNK)
    x1 = xi[:, 1].reshape(NW, NCHUNK, CHUNK)
    x2 = xi[:, 2].reshape(NW, NCHUNK, CHUNK)
    score, sq = _sc_gather_score(x0, x1, x2, W_obj, W_task, W_grasp)
    loss = pl.pallas_call(
        _tc_loss_body,
        out_shape=jax.ShapeDtypeStruct((1, 1), jnp.float32),
        out_specs=pl.BlockSpec(memory_space=pltpu.SMEM),
    )(score.reshape(DIM, DIM), y.reshape(DIM, DIM), sq)
    return loss[0, 0]


# speedup vs baseline: 1.1337x; 1.1337x over previous
import functools

import jax
import jax.numpy as jnp
from jax import lax
from jax.experimental import pallas as pl
from jax.experimental.pallas import tpu as pltpu
from jax.experimental.pallas import tpu_sc as plsc

VOCAB = 100000
DIM = 128
BATCH = 16384
LMBDA = 0.01

NC, NS = 2, 16
NW = NC * NS
BPW = BATCH // NW
NCHUNK = 16
CHUNK = BPW // NCHUNK


_mesh = plsc.VectorSubcoreMesh(core_axis_name="c", subcore_axis_name="s")


@functools.partial(
    pl.kernel,
    out_type=(
        jax.ShapeDtypeStruct((BATCH,), jnp.float32),
        jax.ShapeDtypeStruct((NW, DIM), jnp.float32),
    ),
    mesh=_mesh,
    scratch_types=[
        pltpu.VMEM((NCHUNK, CHUNK), jnp.int32),
        pltpu.VMEM((NCHUNK, CHUNK), jnp.int32),
        pltpu.VMEM((NCHUNK, CHUNK), jnp.int32),
        pltpu.VMEM((2 * CHUNK, DIM), jnp.float32),
        pltpu.VMEM((2 * CHUNK, DIM), jnp.float32),
        pltpu.VMEM((2 * CHUNK, DIM), jnp.float32),
        pltpu.VMEM((BPW,), jnp.float32),
        pltpu.VMEM((DIM,), jnp.float32),
        pltpu.SemaphoreType.DMA,
        pltpu.SemaphoreType.DMA,
    ],
)
def _sc_gather_score(x0, x1, x2, w_obj, w_task, w_grasp,
                     score_out, sq_out,
                     i0, i1, i2, hb, rb, tb, sv, sqv, sem0, sem1):
    c = lax.axis_index("c")
    s = lax.axis_index("s")
    wid = s * NC + c

    pltpu.sync_copy(x0.at[wid], i0)
    pltpu.sync_copy(x1.at[wid], i1)
    pltpu.sync_copy(x2.at[wid], i2)

    lanes = lax.iota(jnp.int32, 16)

    _gdn = lax.GatherDimensionNumbers(
        offset_dims=(), collapsed_slice_dims=(0,), start_index_map=(0,))

    def shuffle(v, idx):
        return lax.gather(v, idx[:, None], dimension_numbers=_gdn,
                          slice_sizes=(1,),
                          mode=lax.GatherScatterMode.PROMISE_IN_BOUNDS)

    def hsum(v):
        for shift in (1, 2, 4, 8):
            v = v + shuffle(v, lanes ^ shift)
        return v

    def issue(j, base, sem):
        pltpu.async_copy(w_obj.at[i0.at[j]], hb.at[pl.ds(base, CHUNK)], sem)
        pltpu.async_copy(w_task.at[i1.at[j]], rb.at[pl.ds(base, CHUNK)], sem)
        pltpu.async_copy(w_grasp.at[i2.at[j]], tb.at[pl.ds(base, CHUNK)], sem)

    def drain(base, sem):
        pltpu.make_async_copy(
            w_obj.at[i0.at[0]], hb.at[pl.ds(base, CHUNK)], sem).wait()
        pltpu.make_async_copy(
            w_task.at[i1.at[0]], rb.at[pl.ds(base, CHUNK)], sem).wait()
        pltpu.make_async_copy(
            w_grasp.at[i2.at[0]], tb.at[pl.ds(base, CHUNK)], sem).wait()

    issue(0, 0, sem0)

    def chunk_fn(j, sq8_in):
        slot = lax.rem(j, 2)
        base = slot * CHUNK

        @pl.when(j + 1 < NCHUNK)
        def _():
            @pl.when(slot == 0)
            def _():
                issue(j + 1, CHUNK, sem1)

            @pl.when(slot == 1)
            def _():
                issue(j + 1, 0, sem0)

        @pl.when(slot == 0)
        def _():
            drain(0, sem0)

        @pl.when(slot == 1)
        def _():
            drain(CHUNK, sem1)

        def blk_fn(b, sq8_):
            sq8_ = list(sq8_)
            scorevec = jnp.zeros((16,), jnp.float32)
            for l in range(16):
                r = base + b * 16 + l
                acc = jnp.zeros((16,), jnp.float32)
                for g in range(8):
                    hv = hb[r, pl.ds(g * 16, 16)]
                    rv = rb[r, pl.ds(g * 16, 16)]
                    tv = tb[r, pl.ds(g * 16, 16)]
                    acc = acc + hv * tv * rv
                    sq8_[g] = sq8_[g] + hv * hv + tv * tv + rv * rv
                scorevec = jnp.where(lanes == l, -hsum(acc), scorevec)
            sv[pl.ds(j * CHUNK + b * 16, 16)] = scorevec
            return tuple(sq8_)

        return lax.fori_loop(0, CHUNK // 16, blk_fn, sq8_in)

    sq8 = tuple(jnp.zeros((16,), jnp.float32) for _ in range(8))
    sq8 = lax.fori_loop(0, NCHUNK, chunk_fn, sq8)

    for g in range(8):
        sqv[pl.ds(g * 16, 16)] = sq8[g]
    pltpu.sync_copy(sqv, sq_out.at[wid])
    pltpu.sync_copy(sv, score_out.at[pl.ds(wid * BPW, BPW)])


def _tc_loss_body(score_ref, y_ref, sq_ref, out_ref):
    z = score_ref[...] * y_ref[...]
    sp = jnp.maximum(z, 0.0) + jnp.log1p(jnp.exp(-jnp.abs(z)))
    regul = jnp.sum(sq_ref[...]) * (1.0 / (BATCH * DIM))
    out_ref[0, 0] = jnp.sum(sp) * (1.0 / BATCH) + LMBDA * regul


def kernel(x, y, W_obj, W_task, W_grasp):
    xi = x.astype(jnp.int32)
    x0 = xi[:, 0].reshape(NW, NCHUNK, CHUNK)
    x1 = xi[:, 1].reshape(NW, NCHUNK, CHUNK)
    x2 = xi[:, 2].reshape(NW, NCHUNK, CHUNK)
    score, sq = _sc_gather_score(x0, x1, x2, W_obj, W_task, W_grasp)
    loss = pl.pallas_call(
        _tc_loss_body,
        out_shape=jax.ShapeDtypeStruct((1, 1), jnp.float32),
        out_specs=pl.BlockSpec(memory_space=pltpu.SMEM),
    )(score.reshape(DIM, DIM), y.reshape(DIM, DIM), sq)
    return loss[0, 0]


# speedup vs baseline: 1.2112x; 1.0684x over previous
import functools

import jax
import jax.numpy as jnp
from jax import lax
from jax.experimental import pallas as pl
from jax.experimental.pallas import tpu as pltpu
from jax.experimental.pallas import tpu_sc as plsc

VOCAB = 100000
DIM = 128
BATCH = 16384
LMBDA = 0.01

NC, NS = 2, 16
NW = NC * NS
BPW = BATCH // NW
NCHUNK = 8
CHUNK = BPW // NCHUNK


_mesh = plsc.VectorSubcoreMesh(core_axis_name="c", subcore_axis_name="s")


@functools.partial(
    pl.kernel,
    out_type=(
        jax.ShapeDtypeStruct((BATCH,), jnp.float32),
        jax.ShapeDtypeStruct((NW, DIM), jnp.float32),
    ),
    mesh=_mesh,
    scratch_types=[
        pltpu.VMEM((NCHUNK, CHUNK), jnp.int32),
        pltpu.VMEM((NCHUNK, CHUNK), jnp.int32),
        pltpu.VMEM((NCHUNK, CHUNK), jnp.int32),
        pltpu.VMEM((2 * CHUNK, DIM), jnp.float32),
        pltpu.VMEM((2 * CHUNK, DIM), jnp.float32),
        pltpu.VMEM((2 * CHUNK, DIM), jnp.float32),
        pltpu.VMEM((BPW,), jnp.float32),
        pltpu.VMEM((DIM,), jnp.float32),
        pltpu.SemaphoreType.DMA,
        pltpu.SemaphoreType.DMA,
    ],
)
def _sc_gather_score(x0, x1, x2, w_obj, w_task, w_grasp,
                     score_out, sq_out,
                     i0, i1, i2, hb, rb, tb, sv, sqv, sem0, sem1):
    c = lax.axis_index("c")
    s = lax.axis_index("s")
    wid = s * NC + c

    pltpu.sync_copy(x0.at[wid], i0)
    pltpu.sync_copy(x1.at[wid], i1)
    pltpu.sync_copy(x2.at[wid], i2)

    lanes = lax.iota(jnp.int32, 16)

    _gdn = lax.GatherDimensionNumbers(
        offset_dims=(), collapsed_slice_dims=(0,), start_index_map=(0,))

    def shuffle(v, idx):
        return lax.gather(v, idx[:, None], dimension_numbers=_gdn,
                          slice_sizes=(1,),
                          mode=lax.GatherScatterMode.PROMISE_IN_BOUNDS)

    def hsum(v):
        for shift in (1, 2, 4, 8):
            v = v + shuffle(v, lanes ^ shift)
        return v

    def issue(j, base, sem):
        pltpu.async_copy(w_obj.at[i0.at[j]], hb.at[pl.ds(base, CHUNK)], sem)
        pltpu.async_copy(w_task.at[i1.at[j]], rb.at[pl.ds(base, CHUNK)], sem)
        pltpu.async_copy(w_grasp.at[i2.at[j]], tb.at[pl.ds(base, CHUNK)], sem)

    def drain(base, sem):
        pltpu.make_async_copy(
            w_obj.at[i0.at[0]], hb.at[pl.ds(base, CHUNK)], sem).wait()
        pltpu.make_async_copy(
            w_task.at[i1.at[0]], rb.at[pl.ds(base, CHUNK)], sem).wait()
        pltpu.make_async_copy(
            w_grasp.at[i2.at[0]], tb.at[pl.ds(base, CHUNK)], sem).wait()

    issue(0, 0, sem0)

    def chunk_fn(j, sq8_in):
        slot = lax.rem(j, 2)
        base = slot * CHUNK

        @pl.when(j + 1 < NCHUNK)
        def _():
            @pl.when(slot == 0)
            def _():
                issue(j + 1, CHUNK, sem1)

            @pl.when(slot == 1)
            def _():
                issue(j + 1, 0, sem0)

        @pl.when(slot == 0)
        def _():
            drain(0, sem0)

        @pl.when(slot == 1)
        def _():
            drain(CHUNK, sem1)

        def blk_fn(b, sq8_):
            sq8_ = list(sq8_)
            scorevec = jnp.zeros((16,), jnp.float32)
            for l in range(16):
                r = base + b * 16 + l
                acc = jnp.zeros((16,), jnp.float32)
                for g in range(8):
                    hv = hb[r, pl.ds(g * 16, 16)]
                    rv = rb[r, pl.ds(g * 16, 16)]
                    tv = tb[r, pl.ds(g * 16, 16)]
                    acc = acc + hv * tv * rv
                    sq8_[g] = sq8_[g] + hv * hv + tv * tv + rv * rv
                scorevec = jnp.where(lanes == l, -hsum(acc), scorevec)
            sv[pl.ds(j * CHUNK + b * 16, 16)] = scorevec
            return tuple(sq8_)

        return lax.fori_loop(0, CHUNK // 16, blk_fn, sq8_in)

    sq8 = tuple(jnp.zeros((16,), jnp.float32) for _ in range(8))
    sq8 = lax.fori_loop(0, NCHUNK, chunk_fn, sq8)

    for g in range(8):
        sqv[pl.ds(g * 16, 16)] = sq8[g]
    pltpu.sync_copy(sqv, sq_out.at[wid])
    pltpu.sync_copy(sv, score_out.at[pl.ds(wid * BPW, BPW)])


def _tc_loss_body(score_ref, y_ref, sq_ref, out_ref):
    z = score_ref[...] * y_ref[...]
    sp = jnp.maximum(z, 0.0) + jnp.log1p(jnp.exp(-jnp.abs(z)))
    regul = jnp.sum(sq_ref[...]) * (1.0 / (BATCH * DIM))
    out_ref[0, 0] = jnp.sum(sp) * (1.0 / BATCH) + LMBDA * regul


def kernel(x, y, W_obj, W_task, W_grasp):
    xi = x.astype(jnp.int32)
    x0 = xi[:, 0].reshape(NW, NCHUNK, CHUNK)
    x1 = xi[:, 1].reshape(NW, NCHUNK, CHUNK)
    x2 = xi[:, 2].reshape(NW, NCHUNK, CHUNK)
    score, sq = _sc_gather_score(x0, x1, x2, W_obj, W_task, W_grasp)
    loss = pl.pallas_call(
        _tc_loss_body,
        out_shape=jax.ShapeDtypeStruct((1, 1), jnp.float32),
        out_specs=pl.BlockSpec(memory_space=pltpu.SMEM),
    )(score.reshape(DIM, DIM), y.reshape(DIM, DIM), sq)
    return loss[0, 0]


# speedup vs baseline: 1.2150x; 1.0031x over previous
import functools

import jax
import jax.numpy as jnp
from jax import lax
from jax.experimental import pallas as pl
from jax.experimental.pallas import tpu as pltpu
from jax.experimental.pallas import tpu_sc as plsc

VOCAB = 100000
DIM = 128
BATCH = 16384
LMBDA = 0.01

NC, NS = 2, 16
NW = NC * NS
BPW = BATCH // NW
NCHUNK = 4
CHUNK = BPW // NCHUNK


_mesh = plsc.VectorSubcoreMesh(core_axis_name="c", subcore_axis_name="s")


@functools.partial(
    pl.kernel,
    out_type=(
        jax.ShapeDtypeStruct((BATCH,), jnp.float32),
        jax.ShapeDtypeStruct((NW, DIM), jnp.float32),
    ),
    mesh=_mesh,
    scratch_types=[
        pltpu.VMEM((NCHUNK, CHUNK), jnp.int32),
        pltpu.VMEM((NCHUNK, CHUNK), jnp.int32),
        pltpu.VMEM((NCHUNK, CHUNK), jnp.int32),
        pltpu.VMEM((2 * CHUNK, DIM), jnp.float32),
        pltpu.VMEM((2 * CHUNK, DIM), jnp.float32),
        pltpu.VMEM((2 * CHUNK, DIM), jnp.float32),
        pltpu.VMEM((BPW,), jnp.float32),
        pltpu.VMEM((DIM,), jnp.float32),
        pltpu.SemaphoreType.DMA,
        pltpu.SemaphoreType.DMA,
    ],
)
def _sc_gather_score(x0, x1, x2, w_obj, w_task, w_grasp,
                     score_out, sq_out,
                     i0, i1, i2, hb, rb, tb, sv, sqv, sem0, sem1):
    c = lax.axis_index("c")
    s = lax.axis_index("s")
    wid = s * NC + c

    pltpu.sync_copy(x0.at[wid], i0)
    pltpu.sync_copy(x1.at[wid], i1)
    pltpu.sync_copy(x2.at[wid], i2)

    lanes = lax.iota(jnp.int32, 16)

    _gdn = lax.GatherDimensionNumbers(
        offset_dims=(), collapsed_slice_dims=(0,), start_index_map=(0,))

    def shuffle(v, idx):
        return lax.gather(v, idx[:, None], dimension_numbers=_gdn,
                          slice_sizes=(1,),
                          mode=lax.GatherScatterMode.PROMISE_IN_BOUNDS)

    def hsum(v):
        for shift in (1, 2, 4, 8):
            v = v + shuffle(v, lanes ^ shift)
        return v

    def issue(j, base, sem):
        pltpu.async_copy(w_obj.at[i0.at[j]], hb.at[pl.ds(base, CHUNK)], sem)
        pltpu.async_copy(w_task.at[i1.at[j]], rb.at[pl.ds(base, CHUNK)], sem)
        pltpu.async_copy(w_grasp.at[i2.at[j]], tb.at[pl.ds(base, CHUNK)], sem)

    def drain(base, sem):
        pltpu.make_async_copy(
            w_obj.at[i0.at[0]], hb.at[pl.ds(base, CHUNK)], sem).wait()
        pltpu.make_async_copy(
            w_task.at[i1.at[0]], rb.at[pl.ds(base, CHUNK)], sem).wait()
        pltpu.make_async_copy(
            w_grasp.at[i2.at[0]], tb.at[pl.ds(base, CHUNK)], sem).wait()

    issue(0, 0, sem0)

    def chunk_fn(j, sq8_in):
        slot = lax.rem(j, 2)
        base = slot * CHUNK

        @pl.when(j + 1 < NCHUNK)
        def _():
            @pl.when(slot == 0)
            def _():
                issue(j + 1, CHUNK, sem1)

            @pl.when(slot == 1)
            def _():
                issue(j + 1, 0, sem0)

        @pl.when(slot == 0)
        def _():
            drain(0, sem0)

        @pl.when(slot == 1)
        def _():
            drain(CHUNK, sem1)

        def blk_fn(b, sq8_):
            sq8_ = list(sq8_)
            scorevec = jnp.zeros((16,), jnp.float32)
            for l in range(16):
                r = base + b * 16 + l
                acc = jnp.zeros((16,), jnp.float32)
                for g in range(8):
                    hv = hb[r, pl.ds(g * 16, 16)]
                    rv = rb[r, pl.ds(g * 16, 16)]
                    tv = tb[r, pl.ds(g * 16, 16)]
                    acc = acc + hv * tv * rv
                    sq8_[g] = sq8_[g] + hv * hv + tv * tv + rv * rv
                scorevec = jnp.where(lanes == l, -hsum(acc), scorevec)
            sv[pl.ds(j * CHUNK + b * 16, 16)] = scorevec
            return tuple(sq8_)

        return lax.fori_loop(0, CHUNK // 16, blk_fn, sq8_in)

    sq8 = tuple(jnp.zeros((16,), jnp.float32) for _ in range(8))
    sq8 = lax.fori_loop(0, NCHUNK, chunk_fn, sq8)

    for g in range(8):
        sqv[pl.ds(g * 16, 16)] = sq8[g]
    pltpu.sync_copy(sqv, sq_out.at[wid])
    pltpu.sync_copy(sv, score_out.at[pl.ds(wid * BPW, BPW)])


def _tc_loss_body(score_ref, y_ref, sq_ref, out_ref):
    z = score_ref[...] * y_ref[...]
    sp = jnp.maximum(z, 0.0) + jnp.log1p(jnp.exp(-jnp.abs(z)))
    regul = jnp.sum(sq_ref[...]) * (1.0 / (BATCH * DIM))
    out_ref[0, 0] = jnp.sum(sp) * (1.0 / BATCH) + LMBDA * regul


def kernel(x, y, W_obj, W_task, W_grasp):
    xi = x.astype(jnp.int32)
    x0 = xi[:, 0].reshape(NW, NCHUNK, CHUNK)
    x1 = xi[:, 1].reshape(NW, NCHUNK, CHUNK)
    x2 = xi[:, 2].reshape(NW, NCHUNK, CHUNK)
    score, sq = _sc_gather_score(x0, x1, x2, W_obj, W_task, W_grasp)
    loss = pl.pallas_call(
        _tc_loss_body,
        out_shape=jax.ShapeDtypeStruct((1, 1), jnp.float32),
        out_specs=pl.BlockSpec(memory_space=pltpu.SMEM),
    )(score.reshape(DIM, DIM), y.reshape(DIM, DIM), sq)
    return loss[0, 0]
